# Initial kernel scaffold; baseline (speedup 1.0000x reference)
import jax
import jax.numpy as jnp
from jax.experimental import pallas as pl


def kernel(interaction_feature, concatenated_node_features, We, be, Wn, bn, Wq, Wk, Wv, Wm, lr_W1, lr_b1, lr_W2, lr_b2, cr_W1, cr_b1, cr_W2, cr_b2, mr_W1, mr_b1, mr_W2, mr_b2, num_obj, object_pairs, num_relation):
    raise NotImplementedError("write your pallas kernel here")


# speedup vs baseline: 4.0931x; 4.0931x over previous
import functools

import jax
import jax.numpy as jnp
from jax.experimental import pallas as pl
from jax.experimental.pallas import tpu as pltpu


def _squat_kernel(if_ref, cnf_ref, We_ref, be_ref, Wn_ref, bn_ref,
                  Wq_ref, Wk_ref, Wv_ref, Wm_ref, W1_ref, b1_ref,
                  W2_ref, b2_ref, idx01_ref, idx10_ref, i0_ref, i1_ref,
                  valid_ref, out_ref, masks_ref, *, bB, N, P, M):
    f32 = jnp.float32
    i32 = jnp.int32
    NN = N * N
    R_E = bB * NN
    R_N = bB * N
    R_P = bB * P

    dot = functools.partial(jnp.dot, preferred_element_type=f32)

    nf = dot(cnf_ref[...], Wn_ref[...]) + bn_ref[...]
    q = dot(nf, Wq_ref[...])
    k = dot(nf, Wk_ref[...])

    ef = dot(if_ref[...], We_ref[...]) + be_ref[...]

    rowE = jax.lax.broadcasted_iota(i32, (R_E, R_N), 0)
    colN = jax.lax.broadcasted_iota(i32, (R_E, R_N), 1)
    b_of = rowE // NN
    i_of = (rowE % NN) // N
    j_of = rowE % N
    Eq = (b_of * N + i_of == colN).astype(f32)
    Ek = (b_of * N + j_of == colN).astype(f32)
    edge = dot(ef, Wv_ref[...]) + dot(Eq, q) + dot(Ek, k)

    logits = jnp.sum(edge * Wm_ref[...], axis=1, keepdims=True)
    masks_ref[...] = jax.nn.sigmoid(logits)

    S = ef + edge
    colE = jax.lax.broadcasted_iota(i32, (R_P, R_E), 1)
    bP = jax.lax.broadcasted_iota(i32, (R_P, R_E), 0) // P
    t01 = idx01_ref[...] + bP * NN
    t10 = idx10_ref[...] + bP * NN
    G = ((colE == t01).astype(f32) + (colE == t10).astype(f32)) * 0.25
    te = dot(G, S)

    colN2 = jax.lax.broadcasted_iota(i32, (R_P, R_N), 1)
    bPn = jax.lax.broadcasted_iota(i32, (R_P, R_N), 0) // P
    G1 = (colN2 == i0_ref[...] + bPn * N).astype(f32)
    G2 = (colN2 == i1_ref[...] + bPn * N).astype(f32)
    n1 = dot(G1, nf)
    n2 = dot(G2, nf)

    ci = jnp.concatenate([n1, n2, te], axis=1) * valid_ref[...]

    h = jax.nn.relu(dot(ci, W1_ref[...]) + b1_ref[...])
    out_ref[...] = dot(h, W2_ref[...]) + b2_ref[...]


def kernel(interaction_feature, concatenated_node_features, We, be, Wn, bn,
           Wq, Wk, Wv, Wm, lr_W1, lr_b1, lr_W2, lr_b2, cr_W1, cr_b1,
           cr_W2, cr_b2, mr_W1, mr_b1, mr_W2, mr_b2, num_obj,
           object_pairs, num_relation):
    B, N, _, De = interaction_feature.shape
    Dn = concatenated_node_features.shape[-1]
    P = object_pairs.shape[1]
    M = We.shape[1]
    H = lr_W1.shape[1]
    NN = N * N
    C_lr = lr_W2.shape[1]
    C_cr = cr_W2.shape[1]
    C_mr = mr_W2.shape[1]
    C = 32
    f32 = jnp.float32

    bB = 4
    grid = B // bB

    if_flat = interaction_feature.reshape(B * NN, De)
    cnf_flat = concatenated_node_features.reshape(B * N, Dn)

    i0 = object_pairs[..., 0].astype(jnp.int32).reshape(B * P, 1)
    i1 = object_pairs[..., 1].astype(jnp.int32).reshape(B * P, 1)
    idx01 = i0 * N + i1
    idx10 = i1 * N + i0
    valid = (jnp.arange(P, dtype=jnp.int32)[None, :]
             < num_relation[:, None]).astype(f32).reshape(B * P, 1)

    W1cat = jnp.concatenate([lr_W1, cr_W1, mr_W1], axis=1)
    b1cat = jnp.concatenate([lr_b1, cr_b1, mr_b1]).reshape(1, 3 * H)
    W2bd = jnp.zeros((3 * H, C), f32)
    W2bd = W2bd.at[:H, :C_lr].set(lr_W2)
    W2bd = W2bd.at[H:2 * H, C_lr:C_lr + C_cr].set(cr_W2)
    W2bd = W2bd.at[2 * H:, C_lr + C_cr:C_lr + C_cr + C_mr].set(mr_W2)
    b2cat = jnp.zeros((1, C), f32)
    b2cat = b2cat.at[0, :C_lr].set(lr_b2)
    b2cat = b2cat.at[0, C_lr:C_lr + C_cr].set(cr_b2)
    b2cat = b2cat.at[0, C_lr + C_cr:C_lr + C_cr + C_mr].set(mr_b2)

    be2 = be.reshape(1, M)
    bn2 = bn.reshape(1, M)
    Wm_row = Wm.reshape(1, M)

    def fixed(shape):
        return pl.BlockSpec(shape, lambda i: (0, 0))

    out_small, masks_flat = pl.pallas_call(
        functools.partial(_squat_kernel, bB=bB, N=N, P=P, M=M),
        grid=(grid,),
        in_specs=[
            pl.BlockSpec((bB * NN, De), lambda i: (i, 0)),
            pl.BlockSpec((bB * N, Dn), lambda i: (i, 0)),
            fixed((De, M)),
            fixed((1, M)),
            fixed((Dn, M)),
            fixed((1, M)),
            fixed((M, M)),
            fixed((M, M)),
            fixed((M, M)),
            fixed((1, M)),
            fixed((3 * M, 3 * H)),
            fixed((1, 3 * H)),
            fixed((3 * H, C)),
            fixed((1, C)),
            pl.BlockSpec((bB * P, 1), lambda i: (i, 0)),
            pl.BlockSpec((bB * P, 1), lambda i: (i, 0)),
            pl.BlockSpec((bB * P, 1), lambda i: (i, 0)),
            pl.BlockSpec((bB * P, 1), lambda i: (i, 0)),
            pl.BlockSpec((bB * P, 1), lambda i: (i, 0)),
        ],
        out_specs=[
            pl.BlockSpec((bB * P, C), lambda i: (i, 0)),
            pl.BlockSpec((bB * NN, 1), lambda i: (i, 0)),
        ],
        out_shape=[
            jax.ShapeDtypeStruct((B * P, C), f32),
            jax.ShapeDtypeStruct((B * NN, 1), f32),
        ],
        compiler_params=pltpu.CompilerParams(
            dimension_semantics=("arbitrary",)),
    )(if_flat, cnf_flat, We, be2, Wn, bn2, Wq, Wk, Wv, Wm_row,
      W1cat, b1cat, W2bd, b2cat, idx01, idx10, i0, i1, valid)

    out = out_small.reshape(B, P, C)
    lr = out[..., :C_lr]
    cr = out[..., C_lr:C_lr + C_cr]
    mr = out[..., C_lr + C_cr:C_lr + C_cr + C_mr]
    masks = masks_flat.reshape(B, N, N)
    return (lr, cr, mr, masks)
